# baseline (speedup 1.0000x reference)
import functools

import jax
import jax.numpy as jnp
from jax import lax
from jax.experimental import pallas as pl
from jax.experimental.pallas import tpu as pltpu
from jax.experimental.pallas import tpu_sc as plsc

N_E = 8192
E_DIM = 256
BETA = 0.25

BM = 1024
BK = 512
NK = N_E // BK


def _process(mm_ref, c2_ref, z2_ref, minv_ref, mini_ref, iota_ref, kk):
    t1 = z2_ref[...] + c2_ref[kk]
    dist = t1 + mm_ref[...]
    lmin = jnp.min(dist, axis=1, keepdims=True)
    cand = jnp.where(dist == lmin, iota_ref[...], float(N_E))
    lidx = jnp.min(cand, axis=1, keepdims=True) + kk.astype(jnp.float32) * BK

    @pl.when(kk == 0)
    def _():
        minv_ref[...] = lmin
        mini_ref[...] = lidx

    @pl.when(kk > 0)
    def _():
        upd = lmin < minv_ref[...]
        minv_ref[...] = jnp.where(upd, lmin, minv_ref[...])
        mini_ref[...] = jnp.where(upd, lidx, mini_ref[...])


def _dist_argmin_body(zfs_ref, cb_ref, idx_ref, loss_ref,
                      mma_ref, mmb_ref, z2_ref, c2_ref, minv_ref, mini_ref,
                      iota_ref):
    m = pl.program_id(0)
    k = pl.program_id(1)
    nm = pl.num_programs(0)

    @pl.when(jnp.logical_and(m == 0, k == 0))
    def _():
        iota_ref[...] = lax.broadcasted_iota(
            jnp.int32, (BM, BK), 1).astype(jnp.float32)

    @pl.when(k == 0)
    def _():
        zfs = zfs_ref[...]
        z2_ref[...] = 0.25 * jnp.sum(zfs * zfs, axis=1, keepdims=True)

    @pl.when(jnp.logical_and(m == 0, k < NK))
    def _():
        cb = cb_ref[...]
        c2_ref[k] = jnp.sum(cb * cb, axis=1)[None, :]

    @pl.when(k % 2 == 0)
    def _():
        @pl.when(k < NK)
        def _():
            mma_ref[...] = lax.dot_general(
                zfs_ref[...], cb_ref[...], (((1,), (1,)), ((), ())),
                preferred_element_type=jnp.float32)

        @pl.when(k > 0)
        def _():
            _process(mmb_ref, c2_ref, z2_ref, minv_ref, mini_ref,
                     iota_ref, k - 1)

    @pl.when(k % 2 == 1)
    def _():
        @pl.when(k < NK)
        def _():
            mmb_ref[...] = lax.dot_general(
                zfs_ref[...], cb_ref[...], (((1,), (1,)), ((), ())),
                preferred_element_type=jnp.float32)

        _process(mma_ref, c2_ref, z2_ref, minv_ref, mini_ref,
                 iota_ref, k - 1)

    @pl.when(k == NK)
    def _():
        idx_ref[...] = mini_ref[...].astype(jnp.int32)
        s = jnp.sum(minv_ref[...])

        @pl.when(m == 0)
        def _():
            loss_ref[0, 0] = s

        @pl.when(m > 0)
        def _():
            loss_ref[0, 0] = loss_ref[0, 0] + s

        @pl.when(m == nm - 1)
        def _():
            scale = (1.0 + BETA) / float(N_E * E_DIM)
            loss_ref[0, 0] = loss_ref[0, 0] * scale


def _dist_argmin(zfs, codebook):
    n_tok = zfs.shape[0]
    grid = (n_tok // BM, NK + 1)
    return pl.pallas_call(
        _dist_argmin_body,
        grid=grid,
        in_specs=[
            pl.BlockSpec((BM, E_DIM), lambda m, k: (m, 0)),
            pl.BlockSpec((BK, E_DIM), lambda m, k: (jnp.minimum(k, NK - 1), 0)),
        ],
        out_specs=[
            pl.BlockSpec((BM, 1), lambda m, k: (m, 0)),
            pl.BlockSpec(memory_space=pltpu.SMEM),
        ],
        out_shape=[
            jax.ShapeDtypeStruct((n_tok, 1), jnp.int32),
            jax.ShapeDtypeStruct((1, 1), jnp.float32),
        ],
        scratch_shapes=[
            pltpu.VMEM((BM, BK), jnp.float32),
            pltpu.VMEM((BM, BK), jnp.float32),
            pltpu.VMEM((BM, 1), jnp.float32),
            pltpu.VMEM((NK, 1, BK), jnp.float32),
            pltpu.VMEM((BM, 1), jnp.float32),
            pltpu.VMEM((BM, 1), jnp.float32),
            pltpu.VMEM((BM, BK), jnp.float32),
        ],
        compiler_params=pltpu.CompilerParams(
            dimension_semantics=("arbitrary", "arbitrary"),
        ),
    )(zfs, codebook)


_NC = 2
_NS = 16
_NW = _NC * _NS
_TOK = 8192
_BPW = _TOK // _NW
_GCH = 128


def _gather_body(cb_hbm, idx_hbm, out_hbm, idx_v, rows_v, sem):
    wid = lax.axis_index("s") * _NC + lax.axis_index("c")
    base = wid * _BPW
    for j in range(_BPW // _GCH):
        pltpu.sync_copy(idx_hbm.at[pl.ds(base + j * _GCH, _GCH)], idx_v.at[j])
        pltpu.async_copy(cb_hbm.at[idx_v.at[j]],
                         rows_v.at[pl.ds(j * _GCH, _GCH)], sem).wait()
    pltpu.sync_copy(rows_v, out_hbm.at[pl.ds(base, _BPW)])


@functools.cache
def _sc_gather_fn():
    return pl.kernel(
        _gather_body,
        out_type=jax.ShapeDtypeStruct((_TOK, E_DIM), jnp.float32),
        mesh=plsc.VectorSubcoreMesh(core_axis_name="c", subcore_axis_name="s"),
        scratch_types=[
            pltpu.VMEM((_BPW // _GCH, _GCH), jnp.int32),
            pltpu.VMEM((_BPW, E_DIM), jnp.float32),
            pltpu.SemaphoreType.DMA,
        ],
    )


def _st_body(z_ref, zq_ref, out_ref):
    z = z_ref[0]
    zqt = zq_ref[...].T
    out_ref[0] = z + (zqt - z)


def _st_transpose(z3, zq):
    b, c, hw = z3.shape
    return pl.pallas_call(
        _st_body,
        grid=(b,),
        in_specs=[
            pl.BlockSpec((1, c, hw), lambda i: (i, 0, 0)),
            pl.BlockSpec((hw, c), lambda i: (i, 0)),
        ],
        out_specs=pl.BlockSpec((1, c, hw), lambda i: (i, 0, 0)),
        out_shape=jax.ShapeDtypeStruct((b, c, hw), jnp.float32),
    )(z3, zq)


def kernel(z, codebook):
    b, c, h, w = z.shape
    zfs = jnp.transpose(z, (0, 2, 3, 1)).reshape(-1, E_DIM) * -2.0
    idx2, loss = _dist_argmin(zfs, codebook)
    idx = idx2.reshape(-1)
    zq = _sc_gather_fn()(codebook, idx)
    z3 = z.reshape(b, c, h * w)
    out3 = _st_transpose(z3, zq)
    z_q_out = out3.reshape(b, c, h, w)
    return (z_q_out, loss[0, 0], idx)


# speedup vs baseline: 1.2601x; 1.2601x over previous
import functools

import jax
import jax.numpy as jnp
from jax import lax
from jax.experimental import pallas as pl
from jax.experimental.pallas import tpu as pltpu
from jax.experimental.pallas import tpu_sc as plsc

N_E = 8192
E_DIM = 256
BETA = 0.25

BM = 1024
BK = 512
NK = N_E // BK


def _dist_argmin_body(zfs_ref, cb_ref, idx_ref, loss_ref,
                      z2_ref, c2_ref, minv_ref, mini_ref, iota_ref):
    m = pl.program_id(0)
    k = pl.program_id(1)
    nm = pl.num_programs(0)

    @pl.when(jnp.logical_and(m == 0, k == 0))
    def _():
        iota_ref[...] = lax.broadcasted_iota(
            jnp.int32, (BM, BK), 1).astype(jnp.float32)

    @pl.when(k == 0)
    def _():
        zfs = zfs_ref[...]
        z2_ref[...] = 0.25 * jnp.sum(zfs * zfs, axis=1, keepdims=True)

    @pl.when(m == 0)
    def _():
        cb = cb_ref[...]
        c2_ref[k] = jnp.sum(cb * cb, axis=1)[None, :]

    mm = lax.dot_general(zfs_ref[...], cb_ref[...], (((1,), (1,)), ((), ())),
                         preferred_element_type=jnp.float32)
    t1 = z2_ref[...] + c2_ref[k]
    dist = t1 + mm
    lmin = jnp.min(dist, axis=1, keepdims=True)
    cand = jnp.where(dist == lmin, iota_ref[...], float(N_E))
    lidx = jnp.min(cand, axis=1, keepdims=True) + k.astype(jnp.float32) * BK

    @pl.when(k == 0)
    def _():
        minv_ref[...] = lmin
        mini_ref[...] = lidx

    @pl.when(k > 0)
    def _():
        upd = lmin < minv_ref[...]
        minv_ref[...] = jnp.where(upd, lmin, minv_ref[...])
        mini_ref[...] = jnp.where(upd, lidx, mini_ref[...])

    @pl.when(k == NK - 1)
    def _():
        idx_ref[...] = mini_ref[...].astype(jnp.int32)
        s = jnp.sum(minv_ref[...])

        @pl.when(m == 0)
        def _():
            loss_ref[0, 0] = s

        @pl.when(m > 0)
        def _():
            loss_ref[0, 0] = loss_ref[0, 0] + s

        @pl.when(m == nm - 1)
        def _():
            scale = (1.0 + BETA) / float(N_E * E_DIM)
            loss_ref[0, 0] = loss_ref[0, 0] * scale


def _dist_argmin(zfs, codebook):
    n_tok = zfs.shape[0]
    grid = (n_tok // BM, NK)
    return pl.pallas_call(
        _dist_argmin_body,
        grid=grid,
        in_specs=[
            pl.BlockSpec((BM, E_DIM), lambda m, k: (m, 0)),
            pl.BlockSpec((BK, E_DIM), lambda m, k: (k, 0)),
        ],
        out_specs=[
            pl.BlockSpec((BM, 1), lambda m, k: (m, 0)),
            pl.BlockSpec(memory_space=pltpu.SMEM),
        ],
        out_shape=[
            jax.ShapeDtypeStruct((n_tok, 1), jnp.int32),
            jax.ShapeDtypeStruct((1, 1), jnp.float32),
        ],
        scratch_shapes=[
            pltpu.VMEM((BM, 1), jnp.float32),
            pltpu.VMEM((NK, 1, BK), jnp.float32),
            pltpu.VMEM((BM, 1), jnp.float32),
            pltpu.VMEM((BM, 1), jnp.float32),
            pltpu.VMEM((BM, BK), jnp.float32),
        ],
        compiler_params=pltpu.CompilerParams(
            dimension_semantics=("arbitrary", "arbitrary"),
        ),
    )(zfs, codebook)


_NC = 2
_NS = 16
_NW = _NC * _NS
_TOK = 8192
_BPW = _TOK // _NW
_GCH = 128


def _gather_body(cb_hbm, idx_hbm, out_hbm, idx_v, rows_v, sem):
    wid = lax.axis_index("s") * _NC + lax.axis_index("c")
    base = wid * _BPW
    for j in range(_BPW // _GCH):
        pltpu.sync_copy(idx_hbm.at[pl.ds(base + j * _GCH, _GCH)], idx_v.at[j])
        pltpu.async_copy(cb_hbm.at[idx_v.at[j]],
                         rows_v.at[pl.ds(j * _GCH, _GCH)], sem).wait()
    pltpu.sync_copy(rows_v, out_hbm.at[pl.ds(base, _BPW)])


@functools.cache
def _sc_gather_fn():
    return pl.kernel(
        _gather_body,
        out_type=jax.ShapeDtypeStruct((_TOK, E_DIM), jnp.float32),
        mesh=plsc.VectorSubcoreMesh(core_axis_name="c", subcore_axis_name="s"),
        scratch_types=[
            pltpu.VMEM((_BPW // _GCH, _GCH), jnp.int32),
            pltpu.VMEM((_BPW, E_DIM), jnp.float32),
            pltpu.SemaphoreType.DMA,
        ],
    )


def _st_body(z_ref, zq_ref, out_ref):
    z = z_ref[0]
    zqt = zq_ref[...].T
    out_ref[0] = z + (zqt - z)


def _st_transpose(z3, zq):
    b, c, hw = z3.shape
    return pl.pallas_call(
        _st_body,
        grid=(b,),
        in_specs=[
            pl.BlockSpec((1, c, hw), lambda i: (i, 0, 0)),
            pl.BlockSpec((hw, c), lambda i: (i, 0)),
        ],
        out_specs=pl.BlockSpec((1, c, hw), lambda i: (i, 0, 0)),
        out_shape=jax.ShapeDtypeStruct((b, c, hw), jnp.float32),
    )(z3, zq)


def kernel(z, codebook):
    b, c, h, w = z.shape
    zfs = jnp.transpose(z, (0, 2, 3, 1)).reshape(-1, E_DIM) * -2.0
    idx2, loss = _dist_argmin(zfs, codebook)
    idx = idx2.reshape(-1)
    zq = _sc_gather_fn()(codebook, idx)
    z3 = z.reshape(b, c, h * w)
    out3 = _st_transpose(z3, zq)
    z_q_out = out3.reshape(b, c, h, w)
    return (z_q_out, loss[0, 0], idx)
